# baseline (speedup 1.0000x reference)
import functools

import jax
import jax.numpy as jnp
from jax import lax
from jax.experimental import pallas as pl
from jax.experimental.pallas import tpu as pltpu
from jax.experimental.pallas import tpu_sc as plsc

_B = 16384
_D = 64
_L = 16
_HALF = 8192
_NC = 2
_NS = 16
_NW = _NC * _NS
_RPW = _B // _NW
_CHUNK = 128
_NCHUNK = _RPW // _CHUNK
_GPC = _CHUNK // _L


def _sc_body(hq, tq, rq, hp, tp, rp, ent_hbm, rel_hbm, out_hbm,
             qh_v, qt_v, qr_v, ph_v, pt_v, pr_v,
             rows_h, rows_t, rows_r, p_v, sem):
    wid = lax.axis_index("s") * _NC + lax.axis_index("c")

    pltpu.sync_copy(hq.at[wid], qh_v)
    pltpu.sync_copy(tq.at[wid], qt_v)
    pltpu.sync_copy(rq.at[wid], qr_v)
    pltpu.sync_copy(hp.at[wid], ph_v)
    pltpu.sync_copy(tp.at[wid], pt_v)
    pltpu.sync_copy(rp.at[wid], pr_v)

    for c in range(_NCHUNK):
        cps = [
            pltpu.async_copy(ent_hbm.at[qh_v.at[c]], rows_h, sem),
            pltpu.async_copy(ent_hbm.at[qt_v.at[c]], rows_t, sem),
            pltpu.async_copy(rel_hbm.at[qr_v.at[c]], rows_r, sem),
        ]
        for cp in cps:
            cp.wait()

        def group(g, _):
            gsl = pl.ds(g * _L, _L)
            pvh = ph_v[c, gsl]
            pvt = pt_v[c, gsl]
            pvr = pr_v[c, gsl]
            for rr in range(_L):
                row = g * _L + rr
                oh = pvh[rr]
                ot = pvt[rr]
                orr = pvr[rr]
                s = None
                for k in range(_D // _L):
                    dh = rows_h[row, pl.ds(oh + k * _L, _L)]
                    dr = rows_r[row, pl.ds(orr + k * _L, _L)]
                    dt = rows_t[row, pl.ds(ot + k * _L, _L)]
                    d = dh + dr - dt
                    sq = d * d
                    s = sq if s is None else s + sq
                p_v[pl.ds((c * _CHUNK + row) * _L, _L)] = s
            return 0

        lax.fori_loop(0, _GPC, group, 0)

    pltpu.sync_copy(p_v, out_hbm.at[pl.ds(wid * _RPW * _L, _RPW * _L)])


@functools.partial(
    pl.kernel,
    out_type=jax.ShapeDtypeStruct((_B * _L,), jnp.float32),
    mesh=plsc.VectorSubcoreMesh(core_axis_name="c", subcore_axis_name="s"),
    scratch_types=[
        pltpu.VMEM((_NCHUNK, _CHUNK), jnp.int32),
        pltpu.VMEM((_NCHUNK, _CHUNK), jnp.int32),
        pltpu.VMEM((_NCHUNK, _CHUNK), jnp.int32),
        pltpu.VMEM((_NCHUNK, _CHUNK), jnp.int32),
        pltpu.VMEM((_NCHUNK, _CHUNK), jnp.int32),
        pltpu.VMEM((_NCHUNK, _CHUNK), jnp.int32),
        pltpu.VMEM((_CHUNK, 2 * _D), jnp.float32),
        pltpu.VMEM((_CHUNK, 2 * _D), jnp.float32),
        pltpu.VMEM((_CHUNK, 2 * _D), jnp.float32),
        pltpu.VMEM((_RPW * _L,), jnp.float32),
        pltpu.SemaphoreType.DMA,
    ],
)
def _transe_partials(hq, tq, rq, hp, tp, rp, ent_hbm, rel_hbm, out_hbm,
                     qh_v, qt_v, qr_v, ph_v, pt_v, pr_v,
                     rows_h, rows_t, rows_r, p_v, sem):
    _sc_body(hq, tq, rq, hp, tp, rp, ent_hbm, rel_hbm, out_hbm,
             qh_v, qt_v, qr_v, ph_v, pt_v, pr_v,
             rows_h, rows_t, rows_r, p_v, sem)


def _fold_body(p_ref, o_ref):
    o_ref[...] = jnp.sqrt(jnp.sum(p_ref[...], axis=-1))


_fold_sqrt = pl.pallas_call(
    _fold_body,
    out_shape=jax.ShapeDtypeStruct((_B,), jnp.float32),
)


def kernel(h, r, t, batch_size, ent_emb, rel_emb):
    del batch_size
    h = h.astype(jnp.int32)
    t = t.astype(jnp.int32)
    r = r.astype(jnp.int32)
    shape3 = (_NW, _NCHUNK, _CHUNK)
    hq = (h >> 1).reshape(shape3)
    tq = (t >> 1).reshape(shape3)
    rq = (r >> 1).reshape(shape3)
    hp = ((h & 1) * _D).reshape(shape3)
    tp = ((t & 1) * _D).reshape(shape3)
    rp = ((r & 1) * _D).reshape(shape3)
    ent2 = ent_emb.reshape(-1, 2 * _D)
    rel2 = rel_emb.reshape(-1, 2 * _D)
    partials = _transe_partials(hq, tq, rq, hp, tp, rp, ent2, rel2)
    score = _fold_sqrt(partials.reshape(_B, _L))
    return score[:_HALF], score[_HALF:]
